# baseline (speedup 1.0000x reference)
import functools

import jax
import jax.numpy as jnp
from jax import lax
from jax.experimental import pallas as pl
from jax.experimental.pallas import tpu as pltpu


def _silu(x):
    return x * jax.nn.sigmoid(x)


def _fused_chain_kernel(x_ref, c0_ref, w1u0_ref, w1m0_ref, w2u0_ref, w2m0_ref,
                        wsk0_ref, b20_ref, c1_ref, w1u1_ref, w1m1_ref,
                        w2u1_ref, w2m1_ref, b21_ref, out_ref, s_ref,
                        *, H, W):
    HW = H * W
    G = 128
    bf = jnp.bfloat16

    idx = lax.broadcasted_iota(jnp.int32, (1, HW), 1)
    col = idx % W
    mxl = (col != 0).astype(bf)
    mxr = (col != W - 1).astype(bf)

    s_ref[:, G - 2 * W:G - W] = jnp.zeros(s_ref.shape[:1] + (W,), bf)
    s_ref[:, G - W + HW:G + HW] = jnp.zeros(s_ref.shape[:1] + (W,), bf)

    def put_windows(act_bf):
        c3 = 3 * act_bf.shape[0]
        v3 = jnp.concatenate(
            [pltpu.roll(act_bf, 1, 1) * mxl, act_bf,
             pltpu.roll(act_bf, HW - 1, 1) * mxr], axis=0)
        s_ref[0:c3, G:G + HW] = v3
        s_ref[c3:2 * c3, G - W:G - W + HW] = v3
        return c3

    def conv3x3(act_bf, wup_ref, wmd_ref):
        c3 = put_windows(act_bf)
        up = s_ref[c3:2 * c3, G - 2 * W:G - 2 * W + HW]
        middn = s_ref[0:2 * c3, G:G + HW]
        return (jnp.dot(wmd_ref[...], middn, preferred_element_type=jnp.float32) +
                jnp.dot(wup_ref[...], up, preferred_element_type=jnp.float32))

    a0 = x_ref[0]

    h = conv3x3(_silu(a0).astype(bf), w1u0_ref, w1m0_ref)
    c0 = c0_ref[0]
    cmid = c0.shape[0] // 2
    h = _silu(c0[:cmid] * h + c0[cmid:])
    a1 = (conv3x3(h.astype(bf), w2u0_ref, w2m0_ref) +
          jnp.dot(wsk0_ref[...], a0.astype(bf),
                  preferred_element_type=jnp.float32))
    a1 = (a1 + b20_ref[...]).astype(bf)

    h = conv3x3(_silu(a1.astype(jnp.float32)).astype(bf), w1u1_ref, w1m1_ref)
    c1 = c1_ref[0]
    h = _silu(c1[:cmid] * h + c1[cmid:])
    h = conv3x3(h.astype(bf), w2u1_ref, w2m1_ref)
    out_ref[0] = a1.astype(jnp.float32) + (h + b21_ref[...])


def kernel(x, time, w1k0, b1k0, wc0, bc0, w2k0, b2k0, wskipk0,
           w1k1, b1k1, wc1, bc1, w2k1, b2k1):
    x = x.astype(jnp.float32)
    B, C0, H, W = x.shape
    HW = H * W
    bf = jnp.bfloat16
    HI = lax.Precision.HIGHEST

    c1out = w1k0.shape[1]

    def wgroups(wk, cin):
        g = (jnp.transpose(wk.reshape(3, 3, c1out, cin), (0, 2, 1, 3))
             .reshape(3, c1out, 3 * cin).astype(bf))
        return g[0], jnp.concatenate([g[1], g[2]], axis=1)

    w1u0, w1m0 = wgroups(w1k0, C0)
    w2u0, w2m0 = wgroups(w2k0, c1out)
    w1u1, w1m1 = wgroups(w1k1, c1out)
    w2u1, w2m1 = wgroups(w2k1, c1out)
    wsk0 = wskipk0.astype(bf)

    def cond_eff(wc, bc, b1):
        c = jnp.dot(time, wc, precision=HI) + bc
        scale, bias = c[:, :c1out], c[:, c1out:]
        return jnp.concatenate([scale, scale * b1.reshape(1, c1out) + bias],
                               axis=1).reshape(B, 2 * c1out, 1)

    c0 = cond_eff(wc0, bc0, b1k0)
    c1 = cond_eff(wc1, bc1, b1k1)

    def full(shape):
        n = len(shape)
        return pl.BlockSpec(shape, lambda b: (0,) * n)

    args = [x.reshape(B, C0, HW), c0, w1u0, w1m0, w2u0, w2m0, wsk0, b2k0,
            c1, w1u1, w1m1, w2u1, w2m1, b2k1]
    in_specs = [pl.BlockSpec((1, C0, HW), lambda b: (b, 0, 0)),
                pl.BlockSpec((1, 2 * c1out, 1), lambda b: (b, 0, 0)),
                full(w1u0.shape), full(w1m0.shape), full(w2u0.shape),
                full(w2m0.shape), full(wsk0.shape), full(b2k0.shape),
                pl.BlockSpec((1, 2 * c1out, 1), lambda b: (b, 0, 0)),
                full(w1u1.shape), full(w1m1.shape), full(w2u1.shape),
                full(w2m1.shape), full(b2k1.shape)]

    out = pl.pallas_call(
        functools.partial(_fused_chain_kernel, H=H, W=W),
        out_shape=jax.ShapeDtypeStruct((B, c1out, HW), jnp.float32),
        grid=(B,),
        in_specs=in_specs,
        out_specs=pl.BlockSpec((1, c1out, HW), lambda b: (b, 0, 0)),
        scratch_shapes=[pltpu.VMEM((6 * c1out, 2 * 128 + HW), bf)],
        compiler_params=pltpu.CompilerParams(
            dimension_semantics=("parallel",)),
    )(*args)
    return out.reshape(B, c1out, H, W)


# speedup vs baseline: 1.1060x; 1.1060x over previous
import functools

import jax
import jax.numpy as jnp
from jax import lax
from jax.experimental import pallas as pl
from jax.experimental.pallas import tpu as pltpu


def _silu(x):
    return x * jax.nn.sigmoid(x)


def _fused_chain_kernel(x_ref, c0_ref, w1g0_ref, w2g0_ref, wsk0_ref, b20_ref,
                        c1_ref, w1g1_ref, w2g1_ref, b21_ref, out_ref, s_ref,
                        *, H, W):
    HW = H * W
    G = 128
    bf = jnp.bfloat16

    idx = lax.broadcasted_iota(jnp.int32, (1, HW), 1)
    col = idx % W
    mxl = (col != 0).astype(bf)
    mxr = (col != W - 1).astype(bf)

    s_ref[:, G - 2 * W:G - W] = jnp.zeros(s_ref.shape[:1] + (W,), bf)
    s_ref[:, G - W + HW:G + HW] = jnp.zeros(s_ref.shape[:1] + (W,), bf)

    def put_windows(act_bf):
        c3 = 3 * act_bf.shape[0]
        v3 = jnp.concatenate(
            [pltpu.roll(act_bf, 1, 1) * mxl, act_bf,
             pltpu.roll(act_bf, HW - 1, 1) * mxr], axis=0)
        s_ref[0:c3, G:G + HW] = v3
        s_ref[c3:2 * c3, G - W:G - W + HW] = v3
        return c3

    def conv3x3(act_bf, wg_ref):
        c3 = put_windows(act_bf)
        up = s_ref[c3:2 * c3, G - 2 * W:G - 2 * W + HW]
        mid = s_ref[0:c3, G:G + HW]
        dn = s_ref[c3:2 * c3, G:G + HW]
        return (jnp.dot(wg_ref[0], up, preferred_element_type=jnp.float32) +
                jnp.dot(wg_ref[1], mid, preferred_element_type=jnp.float32) +
                jnp.dot(wg_ref[2], dn, preferred_element_type=jnp.float32))

    a0 = x_ref[0]

    h = conv3x3(_silu(a0).astype(bf), w1g0_ref)
    c0 = c0_ref[0]
    cmid = c0.shape[0] // 2
    h = _silu(c0[:cmid] * h + c0[cmid:])
    a1 = (conv3x3(h.astype(bf), w2g0_ref) +
          jnp.dot(wsk0_ref[...], a0.astype(bf),
                  preferred_element_type=jnp.float32))
    a1 = (a1 + b20_ref[...]).astype(bf)

    h = conv3x3(_silu(a1.astype(jnp.float32)).astype(bf), w1g1_ref)
    c1 = c1_ref[0]
    h = _silu(c1[:cmid] * h + c1[cmid:])
    h = conv3x3(h.astype(bf), w2g1_ref)
    out_ref[0] = a1.astype(jnp.float32) + (h + b21_ref[...])


def kernel(x, time, w1k0, b1k0, wc0, bc0, w2k0, b2k0, wskipk0,
           w1k1, b1k1, wc1, bc1, w2k1, b2k1):
    x = x.astype(jnp.float32)
    B, C0, H, W = x.shape
    HW = H * W
    bf = jnp.bfloat16
    HI = lax.Precision.HIGHEST

    c1out = w1k0.shape[1]

    def wgroups(wk, cin):
        return (jnp.transpose(wk.reshape(3, 3, c1out, cin), (0, 2, 1, 3))
                .reshape(3, c1out, 3 * cin).astype(bf))

    w1g0 = wgroups(w1k0, C0)
    w2g0 = wgroups(w2k0, c1out)
    w1g1 = wgroups(w1k1, c1out)
    w2g1 = wgroups(w2k1, c1out)
    wsk0 = wskipk0.astype(bf)

    def cond_eff(wc, bc, b1):
        c = jnp.dot(time, wc, precision=HI) + bc
        scale, bias = c[:, :c1out], c[:, c1out:]
        return jnp.concatenate([scale, scale * b1.reshape(1, c1out) + bias],
                               axis=1).reshape(B, 2 * c1out, 1)

    c0 = cond_eff(wc0, bc0, b1k0)
    c1 = cond_eff(wc1, bc1, b1k1)

    def full(shape):
        n = len(shape)
        return pl.BlockSpec(shape, lambda b: (0,) * n)

    args = [x.reshape(B, C0, HW), c0, w1g0, w2g0, wsk0, b2k0,
            c1, w1g1, w2g1, b2k1]
    in_specs = [pl.BlockSpec((1, C0, HW), lambda b: (b, 0, 0)),
                pl.BlockSpec((1, 2 * c1out, 1), lambda b: (b, 0, 0)),
                full(w1g0.shape), full(w2g0.shape), full(wsk0.shape),
                full(b2k0.shape),
                pl.BlockSpec((1, 2 * c1out, 1), lambda b: (b, 0, 0)),
                full(w1g1.shape), full(w2g1.shape), full(b2k1.shape)]

    out = pl.pallas_call(
        functools.partial(_fused_chain_kernel, H=H, W=W),
        out_shape=jax.ShapeDtypeStruct((B, c1out, HW), jnp.float32),
        grid=(B,),
        in_specs=in_specs,
        out_specs=pl.BlockSpec((1, c1out, HW), lambda b: (b, 0, 0)),
        scratch_shapes=[pltpu.VMEM((6 * c1out, 2 * 128 + HW), bf)],
        compiler_params=pltpu.CompilerParams(
            dimension_semantics=("parallel",)),
    )(*args)
    return out.reshape(B, c1out, H, W)
